# baseline (speedup 1.0000x reference)
import functools

import jax
import jax.numpy as jnp
from jax import lax
from jax.experimental import pallas as pl
from jax.experimental.pallas import tpu as pltpu
from jax.experimental.pallas import tpu_sc as plsc

N_DATA = 1000000
EMBED_DIM = 32
HIDDENS = 64
OUTPUT_DIM = 16
BATCH = 16384

_NC = 2
_NS = 16
_NW = _NC * _NS
_B_PER_W = BATCH // _NW


@functools.lru_cache(maxsize=None)
def _make_sc_gather():
  mesh = plsc.VectorSubcoreMesh(core_axis_name="c", subcore_axis_name="s")

  @functools.partial(
      pl.kernel,
      mesh=mesh,
      out_type=jax.ShapeDtypeStruct((BATCH, EMBED_DIM), jnp.float32),
      scratch_types=[
          pltpu.VMEM((_B_PER_W,), jnp.int32),
          pltpu.VMEM((_B_PER_W, EMBED_DIM), jnp.float32),
          pltpu.SemaphoreType.DMA,
      ],
      compiler_params=pltpu.CompilerParams(use_tc_tiling_on_sc=False),
  )
  def gather_kernel(table_hbm, idx_hbm, out_hbm, idx_v, rows_v, sem):
    wid = lax.axis_index("s") * _NC + lax.axis_index("c")
    base = wid * _B_PER_W
    pltpu.sync_copy(idx_hbm.at[pl.ds(base, _B_PER_W)], idx_v)
    pltpu.async_copy(table_hbm.at[idx_v], rows_v, sem).wait()
    pltpu.sync_copy(rows_v, out_hbm.at[pl.ds(base, _B_PER_W)])

  return gather_kernel


_TK = 8192


def _tr_body(in_ref, out_ref):
  out_ref[...] = jnp.swapaxes(in_ref[...], 0, 1)


def _transpose_table(tableT):
  grid = (-(-N_DATA // _TK),)
  return pl.pallas_call(
      _tr_body,
      grid=grid,
      in_specs=[pl.BlockSpec((EMBED_DIM, _TK), lambda i: (0, i))],
      out_specs=pl.BlockSpec((_TK, EMBED_DIM), lambda i: (i, 0)),
      out_shape=jax.ShapeDtypeStruct((N_DATA, EMBED_DIM), jnp.float32),
  )(tableT)


_BM = 2048


def _mlp_body(emb_ref, w1_ref, b1_ref, w2_ref, b2_ref, out_ref):
  emb = emb_ref[...]
  h = jnp.tanh(
      jnp.dot(emb, w1_ref[...], preferred_element_type=jnp.float32)
      + b1_ref[...]
  )
  z = (
      jnp.dot(h, w2_ref[...], preferred_element_type=jnp.float32)
      + b2_ref[...]
  )
  out_ref[...] = jax.nn.sigmoid(z)


def _mlp(emb, w1, b1, w2, b2):
  grid = (BATCH // _BM,)
  return pl.pallas_call(
      _mlp_body,
      grid=grid,
      in_specs=[
          pl.BlockSpec((_BM, EMBED_DIM), lambda i: (i, 0)),
          pl.BlockSpec((EMBED_DIM, HIDDENS), lambda i: (0, 0)),
          pl.BlockSpec((1, HIDDENS), lambda i: (0, 0)),
          pl.BlockSpec((HIDDENS, OUTPUT_DIM), lambda i: (0, 0)),
          pl.BlockSpec((1, OUTPUT_DIM), lambda i: (0, 0)),
      ],
      out_specs=pl.BlockSpec((_BM, OUTPUT_DIM), lambda i: (i, 0)),
      out_shape=jax.ShapeDtypeStruct((BATCH, OUTPUT_DIM), jnp.float32),
  )(emb, w1, b1, w2, b2)


@jax.jit
def kernel(idx, table, W1, b1, W2, b2):
  table_rm = _transpose_table(table.T)
  emb = _make_sc_gather()(table_rm, idx.astype(jnp.int32))
  return _mlp(emb, W1, b1.reshape(1, HIDDENS), W2, b2.reshape(1, OUTPUT_DIM))


# speedup vs baseline: 1.0519x; 1.0519x over previous
import functools

import jax
import jax.numpy as jnp
from jax import lax
from jax.experimental import pallas as pl
from jax.experimental.pallas import tpu as pltpu
from jax.experimental.pallas import tpu_sc as plsc

N_DATA = 1000000
EMBED_DIM = 32
HIDDENS = 64
OUTPUT_DIM = 16
BATCH = 16384

_NC = 2
_NS = 16
_NW = _NC * _NS
_B_PER_W = BATCH // _NW


@functools.lru_cache(maxsize=None)
def _make_sc_gather():
  mesh = plsc.VectorSubcoreMesh(core_axis_name="c", subcore_axis_name="s")

  @functools.partial(
      pl.kernel,
      mesh=mesh,
      out_type=jax.ShapeDtypeStruct((BATCH, EMBED_DIM), jnp.float32),
      scratch_types=[
          pltpu.VMEM((_B_PER_W,), jnp.int32),
          pltpu.VMEM((_B_PER_W, EMBED_DIM), jnp.float32),
          pltpu.SemaphoreType.DMA,
      ],
      compiler_params=pltpu.CompilerParams(use_tc_tiling_on_sc=False),
  )
  def gather_kernel(table_hbm, idx_hbm, out_hbm, idx_v, rows_v, sem):
    wid = lax.axis_index("s") * _NC + lax.axis_index("c")
    base = wid * _B_PER_W
    pltpu.sync_copy(idx_hbm.at[pl.ds(base, _B_PER_W)], idx_v)
    pltpu.async_copy(table_hbm.at[idx_v], rows_v, sem).wait()
    pltpu.sync_copy(rows_v, out_hbm.at[pl.ds(base, _B_PER_W)])

  return gather_kernel


_TK = 16384


def _tr_body(in_ref, out_ref):
  out_ref[...] = jnp.swapaxes(in_ref[...], 0, 1)


def _transpose_table(tableT):
  grid = (-(-N_DATA // _TK),)
  return pl.pallas_call(
      _tr_body,
      grid=grid,
      in_specs=[pl.BlockSpec((EMBED_DIM, _TK), lambda i: (0, i))],
      out_specs=pl.BlockSpec((_TK, EMBED_DIM), lambda i: (i, 0)),
      out_shape=jax.ShapeDtypeStruct((N_DATA, EMBED_DIM), jnp.float32),
  )(tableT)


_BM = 2048


def _mlp_body(emb_ref, w1_ref, b1_ref, w2_ref, b2_ref, out_ref):
  emb = emb_ref[...]
  h = jnp.tanh(
      jnp.dot(emb, w1_ref[...], preferred_element_type=jnp.float32)
      + b1_ref[...]
  )
  z = (
      jnp.dot(h, w2_ref[...], preferred_element_type=jnp.float32)
      + b2_ref[...]
  )
  out_ref[...] = jax.nn.sigmoid(z)


def _mlp(emb, w1, b1, w2, b2):
  grid = (BATCH // _BM,)
  return pl.pallas_call(
      _mlp_body,
      grid=grid,
      in_specs=[
          pl.BlockSpec((_BM, EMBED_DIM), lambda i: (i, 0)),
          pl.BlockSpec((EMBED_DIM, HIDDENS), lambda i: (0, 0)),
          pl.BlockSpec((1, HIDDENS), lambda i: (0, 0)),
          pl.BlockSpec((HIDDENS, OUTPUT_DIM), lambda i: (0, 0)),
          pl.BlockSpec((1, OUTPUT_DIM), lambda i: (0, 0)),
      ],
      out_specs=pl.BlockSpec((_BM, OUTPUT_DIM), lambda i: (i, 0)),
      out_shape=jax.ShapeDtypeStruct((BATCH, OUTPUT_DIM), jnp.float32),
  )(emb, w1, b1, w2, b2)


@jax.jit
def kernel(idx, table, W1, b1, W2, b2):
  table_rm = _transpose_table(table.T)
  emb = _make_sc_gather()(table_rm, idx.astype(jnp.int32))
  return _mlp(emb, W1, b1.reshape(1, HIDDENS), W2, b2.reshape(1, OUTPUT_DIM))


# speedup vs baseline: 1.2940x; 1.2302x over previous
import functools

import jax
import jax.numpy as jnp
from jax import lax
from jax.experimental import pallas as pl
from jax.experimental.pallas import tpu as pltpu
from jax.experimental.pallas import tpu_sc as plsc

N_DATA = 1000000
EMBED_DIM = 32
HIDDENS = 64
OUTPUT_DIM = 16
BATCH = 16384

_NC = 2
_NS = 16
_NW = _NC * _NS
_B_PER_W = BATCH // _NW

_TW = 8192
_TGRID = 122
_MAIN = _TW * _TGRID
_TAIL = N_DATA - _MAIN
_TLINES = _TW * EMBED_DIM // 128
_LINES = _TGRID * _TLINES
_QW = _TW // 4


def _pack_body(in_hbm, out_ref, buf, sem):
  i = pl.program_id(0)

  def start(step, slot):
    pltpu.make_async_copy(
        in_hbm.at[:, pl.ds(step * _TW, _TW)], buf.at[slot], sem.at[slot]
    ).start()

  @pl.when(i == 0)
  def _():
    start(0, 0)

  @pl.when(i + 1 < _TGRID)
  def _():
    start(i + 1, (i + 1) % 2)

  slot = i % 2
  pltpu.make_async_copy(
      in_hbm.at[:, pl.ds(i * _TW, _TW)], buf.at[slot], sem.at[slot]
  ).wait()
  x = buf[slot]
  for q in range(4):
    out_ref[:, EMBED_DIM * q:EMBED_DIM * (q + 1)] = (
        x[:, _QW * q:_QW * (q + 1)].T
    )


def _pack_table(tableT):
  return pl.pallas_call(
      _pack_body,
      grid=(_TGRID,),
      in_specs=[pl.BlockSpec(memory_space=pltpu.MemorySpace.HBM)],
      out_specs=pl.BlockSpec((_TLINES, 128), lambda i: (i, 0)),
      out_shape=jax.ShapeDtypeStruct((_LINES, 128), jnp.float32),
      scratch_shapes=[
          pltpu.VMEM((2, EMBED_DIM, _TW), jnp.float32),
          pltpu.SemaphoreType.DMA((2,)),
      ],
  )(tableT)


@functools.lru_cache(maxsize=None)
def _make_sc_gather():
  mesh = plsc.VectorSubcoreMesh(core_axis_name="c", subcore_axis_name="s")

  @functools.partial(
      pl.kernel,
      mesh=mesh,
      out_type=(
          jax.ShapeDtypeStruct((BATCH, 128), jnp.float32),
          jax.ShapeDtypeStruct((BATCH, EMBED_DIM), jnp.float32),
      ),
      scratch_types=[
          pltpu.VMEM((_B_PER_W,), jnp.int32),
          pltpu.VMEM((_B_PER_W,), jnp.int32),
          pltpu.VMEM((_B_PER_W, 128), jnp.float32),
          pltpu.VMEM((_B_PER_W, EMBED_DIM), jnp.float32),
          pltpu.SemaphoreType.DMA,
          pltpu.SemaphoreType.DMA,
      ],
      compiler_params=pltpu.CompilerParams(use_tc_tiling_on_sc=False),
  )
  def gather_kernel(lines_hbm, tail_hbm, idxm_hbm, idxt_hbm, outm_hbm,
                    outt_hbm, idxm_v, idxt_v, rowsm_v, rowst_v, semm, semt):
    wid = lax.axis_index("s") * _NC + lax.axis_index("c")
    base = wid * _B_PER_W
    pltpu.sync_copy(idxm_hbm.at[pl.ds(base, _B_PER_W)], idxm_v)
    pltpu.sync_copy(idxt_hbm.at[pl.ds(base, _B_PER_W)], idxt_v)
    cm = pltpu.async_copy(lines_hbm.at[idxm_v], rowsm_v, semm)
    ct = pltpu.async_copy(tail_hbm.at[idxt_v], rowst_v, semt)
    cm.wait()
    ct.wait()
    pltpu.sync_copy(rowsm_v, outm_hbm.at[pl.ds(base, _B_PER_W)])
    pltpu.sync_copy(rowst_v, outt_hbm.at[pl.ds(base, _B_PER_W)])

  return gather_kernel


_BM = 2048


def _mlp_body(lines_ref, tail_ref, q_ref, sel_ref, w1_ref, b1_ref, w2_ref,
              b2_ref, out_ref):
  lines = lines_ref[...]
  qv = q_ref[...]
  emb = lines[:, 0:EMBED_DIM]
  for q in range(1, 4):
    emb = jnp.where(
        qv == q, lines[:, EMBED_DIM * q:EMBED_DIM * (q + 1)], emb
    )
  emb = jnp.where(sel_ref[...] > 0, tail_ref[...], emb)
  h = jnp.tanh(
      jnp.dot(emb, w1_ref[...], preferred_element_type=jnp.float32)
      + b1_ref[...]
  )
  z = (
      jnp.dot(h, w2_ref[...], preferred_element_type=jnp.float32)
      + b2_ref[...]
  )
  out_ref[...] = jax.nn.sigmoid(z)


def _mlp(lines, tail, qsel, sel, w1, b1, w2, b2):
  grid = (BATCH // _BM,)
  return pl.pallas_call(
      _mlp_body,
      grid=grid,
      in_specs=[
          pl.BlockSpec((_BM, 128), lambda i: (i, 0)),
          pl.BlockSpec((_BM, EMBED_DIM), lambda i: (i, 0)),
          pl.BlockSpec((_BM, 1), lambda i: (i, 0)),
          pl.BlockSpec((_BM, 1), lambda i: (i, 0)),
          pl.BlockSpec((EMBED_DIM, HIDDENS), lambda i: (0, 0)),
          pl.BlockSpec((1, HIDDENS), lambda i: (0, 0)),
          pl.BlockSpec((HIDDENS, OUTPUT_DIM), lambda i: (0, 0)),
          pl.BlockSpec((1, OUTPUT_DIM), lambda i: (0, 0)),
      ],
      out_specs=pl.BlockSpec((_BM, OUTPUT_DIM), lambda i: (i, 0)),
      out_shape=jax.ShapeDtypeStruct((BATCH, OUTPUT_DIM), jnp.float32),
  )(lines, tail, qsel, sel, w1, b1, w2, b2)


@jax.jit
def kernel(idx, table, W1, b1, W2, b2):
  idx = idx.astype(jnp.int32)
  packed = _pack_table(table.T)
  tail = lax.slice(table, (_MAIN, 0), (N_DATA, EMBED_DIM))
  idx_m = jnp.minimum(idx, _MAIN - 1)
  c = idx_m % _TW
  line_idx = (idx_m // _TW) * _TLINES + (c % _QW)
  qsel = (c // _QW).reshape(BATCH, 1)
  idx_t = jnp.clip(idx - _MAIN, 0, _TAIL - 1)
  lines, emb_t = _make_sc_gather()(packed, tail, line_idx, idx_t)
  sel = (idx >= _MAIN).astype(jnp.int32).reshape(BATCH, 1)
  return _mlp(lines, emb_t, qsel, sel, W1, b1.reshape(1, HIDDENS), W2,
              b2.reshape(1, OUTPUT_DIM))


# speedup vs baseline: 1.3196x; 1.0198x over previous
import functools

import jax
import jax.numpy as jnp
from jax import lax
from jax.experimental import pallas as pl
from jax.experimental.pallas import tpu as pltpu
from jax.experimental.pallas import tpu_sc as plsc

N_DATA = 1000000
EMBED_DIM = 32
HIDDENS = 64
OUTPUT_DIM = 16
BATCH = 16384

_NC = 2
_NS = 16
_NW = _NC * _NS
_B_PER_W = BATCH // _NW

_TW = 8192
_TGRID = 122
_MAIN = _TW * _TGRID
_TAIL = N_DATA - _MAIN
_TLINES = _TW * EMBED_DIM // 128
_LINES = _TGRID * _TLINES
_QW = _TW // 4


def _pack_body(in_hbm, out_ref, buf, sem):
  i = pl.program_id(0)

  def start(step, slot):
    pltpu.make_async_copy(
        in_hbm.at[:, pl.ds(step * _TW, _TW)], buf.at[slot], sem.at[slot]
    ).start()

  @pl.when(i == 0)
  def _():
    start(0, 0)

  @pl.when(i + 1 < _TGRID)
  def _():
    start(i + 1, (i + 1) % 2)

  slot = i % 2
  pltpu.make_async_copy(
      in_hbm.at[:, pl.ds(i * _TW, _TW)], buf.at[slot], sem.at[slot]
  ).wait()
  x = buf[slot]
  for q in range(4):
    out_ref[:, EMBED_DIM * q:EMBED_DIM * (q + 1)] = (
        x[:, _QW * q:_QW * (q + 1)].T
    )


def _pack_table(tableT):
  return pl.pallas_call(
      _pack_body,
      grid=(_TGRID,),
      in_specs=[pl.BlockSpec(memory_space=pltpu.MemorySpace.HBM)],
      out_specs=pl.BlockSpec((_TLINES, 128), lambda i: (i, 0)),
      out_shape=jax.ShapeDtypeStruct((_LINES, 128), jnp.float32),
      scratch_shapes=[
          pltpu.VMEM((2, EMBED_DIM, _TW), jnp.float32),
          pltpu.SemaphoreType.DMA((2,)),
      ],
  )(tableT)


@functools.lru_cache(maxsize=None)
def _make_sc_gather():
  mesh = plsc.VectorSubcoreMesh(core_axis_name="c", subcore_axis_name="s")

  @functools.partial(
      pl.kernel,
      mesh=mesh,
      out_type=(
          jax.ShapeDtypeStruct((BATCH, EMBED_DIM), jnp.float32),
          jax.ShapeDtypeStruct((BATCH, EMBED_DIM), jnp.float32),
      ),
      scratch_types=[
          pltpu.VMEM((_B_PER_W,), jnp.int32),
          pltpu.VMEM((_B_PER_W,), jnp.int32),
          pltpu.VMEM((_B_PER_W, EMBED_DIM), jnp.float32),
          pltpu.VMEM((_B_PER_W, EMBED_DIM), jnp.float32),
          pltpu.SemaphoreType.DMA,
          pltpu.SemaphoreType.DMA,
      ],
      compiler_params=pltpu.CompilerParams(use_tc_tiling_on_sc=False),
  )
  def gather_kernel(main_hbm, tail_hbm, idxm_hbm, idxt_hbm, outm_hbm,
                    outt_hbm, idxm_v, idxt_v, rowsm_v, rowst_v, semm, semt):
    wid = lax.axis_index("s") * _NC + lax.axis_index("c")
    base = wid * _B_PER_W
    pltpu.sync_copy(idxm_hbm.at[pl.ds(base, _B_PER_W)], idxm_v)
    pltpu.sync_copy(idxt_hbm.at[pl.ds(base, _B_PER_W)], idxt_v)
    cm = pltpu.async_copy(main_hbm.at[idxm_v], rowsm_v, semm)
    ct = pltpu.async_copy(tail_hbm.at[idxt_v], rowst_v, semt)
    cm.wait()
    ct.wait()
    pltpu.sync_copy(rowsm_v, outm_hbm.at[pl.ds(base, _B_PER_W)])
    pltpu.sync_copy(rowst_v, outt_hbm.at[pl.ds(base, _B_PER_W)])

  return gather_kernel


_BM = 2048


def _mlp_body(embm_ref, embt_ref, sel_ref, w1_ref, b1_ref, w2_ref,
              b2_ref, out_ref):
  emb = jnp.where(sel_ref[...] > 0, embt_ref[...], embm_ref[...])
  h = jnp.tanh(
      jnp.dot(emb, w1_ref[...], preferred_element_type=jnp.float32)
      + b1_ref[...]
  )
  z = (
      jnp.dot(h, w2_ref[...], preferred_element_type=jnp.float32)
      + b2_ref[...]
  )
  out_ref[...] = jax.nn.sigmoid(z)


def _mlp(embm, embt, sel, w1, b1, w2, b2):
  grid = (BATCH // _BM,)
  return pl.pallas_call(
      _mlp_body,
      grid=grid,
      in_specs=[
          pl.BlockSpec((_BM, EMBED_DIM), lambda i: (i, 0)),
          pl.BlockSpec((_BM, EMBED_DIM), lambda i: (i, 0)),
          pl.BlockSpec((_BM, 1), lambda i: (i, 0)),
          pl.BlockSpec((EMBED_DIM, HIDDENS), lambda i: (0, 0)),
          pl.BlockSpec((1, HIDDENS), lambda i: (0, 0)),
          pl.BlockSpec((HIDDENS, OUTPUT_DIM), lambda i: (0, 0)),
          pl.BlockSpec((1, OUTPUT_DIM), lambda i: (0, 0)),
      ],
      out_specs=pl.BlockSpec((_BM, OUTPUT_DIM), lambda i: (i, 0)),
      out_shape=jax.ShapeDtypeStruct((BATCH, OUTPUT_DIM), jnp.float32),
  )(embm, embt, sel, w1, b1, w2, b2)


@jax.jit
def kernel(idx, table, W1, b1, W2, b2):
  idx = idx.astype(jnp.int32)
  packed = _pack_table(table.T).reshape(_MAIN, EMBED_DIM)
  tail = lax.slice(table, (_MAIN, 0), (N_DATA, EMBED_DIM))
  idx_m = jnp.minimum(idx, _MAIN - 1)
  c = idx_m % _TW
  perm_idx = (idx_m // _TW) * _TW + (c % _QW) * 4 + c // _QW
  idx_t = jnp.clip(idx - _MAIN, 0, _TAIL - 1)
  emb_m, emb_t = _make_sc_gather()(packed, tail, perm_idx, idx_t)
  sel = (idx >= _MAIN).astype(jnp.int32).reshape(BATCH, 1)
  return _mlp(emb_m, emb_t, sel, W1, b1.reshape(1, HIDDENS), W2,
              b2.reshape(1, OUTPUT_DIM))


# speedup vs baseline: 3.0822x; 2.3358x over previous
import functools

import jax
import jax.numpy as jnp
from jax import lax
from jax.experimental import pallas as pl
from jax.experimental.pallas import tpu as pltpu
from jax.experimental.pallas import tpu_sc as plsc

N_DATA = 1000000
EMBED_DIM = 32
HIDDENS = 64
OUTPUT_DIM = 16
BATCH = 16384

_NC = 2
_NS = 16
_NW = _NC * _NS
_B_PER_W = BATCH // _NW

_TW = 16384
_TGRID = 61
_MAIN = _TW * _TGRID
_TAIL = N_DATA - _MAIN
_TLINES = _TW * EMBED_DIM // 128
_QW = _TW // 4
_TAILQ = _TAIL // 4
_LINES = N_DATA * EMBED_DIM // 128


def _pack_body(in_hbm, tail_ref, out_ref, buf, sem):
  i = pl.program_id(0)

  def start(step, slot):
    pltpu.make_async_copy(
        in_hbm.at[:, pl.ds(step * _TW, _TW)], buf.at[slot], sem.at[slot]
    ).start()

  @pl.when(i == 0)
  def _():
    start(0, 0)

  @pl.when(i + 1 < _TGRID)
  def _():
    start(i + 1, (i + 1) % 2)

  slot = i % 2

  @pl.when(i < _TGRID)
  def _():
    pltpu.make_async_copy(
        in_hbm.at[:, pl.ds(0, _TW)], buf.at[slot], sem.at[slot]
    ).wait()
    rows = lax.broadcasted_iota(jnp.int32, (EMBED_DIM, 128), 0)
    cols = lax.broadcasted_iota(jnp.int32, (EMBED_DIM, 128), 1)
    x = buf[slot]
    acc = None
    for q in range(4):
      sel = (rows + EMBED_DIM * q == cols).astype(jnp.float32)
      y = lax.dot_general(
          x[:, _QW * q:_QW * (q + 1)], sel,
          (((0,), (0,)), ((), ())),
          preferred_element_type=jnp.float32,
      )
      acc = y if acc is None else acc + y
    out_ref[...] = acc

  @pl.when(i == _TGRID)
  def _():
    for q in range(4):
      out_ref[0:_TAILQ, EMBED_DIM * q:EMBED_DIM * (q + 1)] = (
          tail_ref[_TAILQ * q:_TAILQ * (q + 1), :]
      )


def _pack_table(tableT, tail):
  return pl.pallas_call(
      _pack_body,
      grid=(_TGRID + 1,),
      in_specs=[
          pl.BlockSpec(memory_space=pltpu.MemorySpace.HBM),
          pl.BlockSpec((_TAIL, EMBED_DIM), lambda i: (0, 0)),
      ],
      out_specs=pl.BlockSpec((_TLINES, 128), lambda i: (i, 0)),
      out_shape=jax.ShapeDtypeStruct((_LINES, 128), jnp.float32),
      scratch_shapes=[
          pltpu.VMEM((2, EMBED_DIM, _TW), jnp.float32),
          pltpu.SemaphoreType.DMA((2,)),
      ],
  )(tableT, tail)


@functools.lru_cache(maxsize=None)
def _make_sc_gather():
  mesh = plsc.VectorSubcoreMesh(core_axis_name="c", subcore_axis_name="s")

  @functools.partial(
      pl.kernel,
      mesh=mesh,
      out_type=jax.ShapeDtypeStruct((BATCH, EMBED_DIM), jnp.float32),
      scratch_types=[
          pltpu.VMEM((_B_PER_W,), jnp.int32),
          pltpu.VMEM((_B_PER_W, EMBED_DIM), jnp.float32),
          pltpu.SemaphoreType.DMA,
      ],
      compiler_params=pltpu.CompilerParams(use_tc_tiling_on_sc=False),
  )
  def gather_kernel(table_hbm, idx_hbm, out_hbm, idx_v, rows_v, sem):
    wid = lax.axis_index("s") * _NC + lax.axis_index("c")
    base = wid * _B_PER_W
    pltpu.sync_copy(idx_hbm.at[pl.ds(base, _B_PER_W)], idx_v)
    pltpu.async_copy(table_hbm.at[idx_v], rows_v, sem).wait()
    pltpu.sync_copy(rows_v, out_hbm.at[pl.ds(base, _B_PER_W)])

  return gather_kernel


_BM = 2048


def _mlp_body(emb_ref, w1_ref, b1_ref, w2_ref, b2_ref, out_ref):
  h = jnp.tanh(
      jnp.dot(emb_ref[...], w1_ref[...], preferred_element_type=jnp.float32)
      + b1_ref[...]
  )
  z = (
      jnp.dot(h, w2_ref[...], preferred_element_type=jnp.float32)
      + b2_ref[...]
  )
  out_ref[...] = jax.nn.sigmoid(z)


def _mlp(emb, w1, b1, w2, b2):
  grid = (BATCH // _BM,)
  return pl.pallas_call(
      _mlp_body,
      grid=grid,
      in_specs=[
          pl.BlockSpec((_BM, EMBED_DIM), lambda i: (i, 0)),
          pl.BlockSpec((EMBED_DIM, HIDDENS), lambda i: (0, 0)),
          pl.BlockSpec((1, HIDDENS), lambda i: (0, 0)),
          pl.BlockSpec((HIDDENS, OUTPUT_DIM), lambda i: (0, 0)),
          pl.BlockSpec((1, OUTPUT_DIM), lambda i: (0, 0)),
      ],
      out_specs=pl.BlockSpec((_BM, OUTPUT_DIM), lambda i: (i, 0)),
      out_shape=jax.ShapeDtypeStruct((BATCH, OUTPUT_DIM), jnp.float32),
  )(emb, w1, b1, w2, b2)


@jax.jit
def kernel(idx, table, W1, b1, W2, b2):
  idx = idx.astype(jnp.int32)
  tail = lax.slice(table, (_MAIN, 0), (N_DATA, EMBED_DIM))
  packed = _pack_table(table.T, tail).reshape(N_DATA, EMBED_DIM)
  c = idx % _TW
  perm_main = (idx // _TW) * _TW + (c % _QW) * 4 + c // _QW
  t = idx - _MAIN
  perm_tail = _MAIN + (t % _TAILQ) * 4 + t // _TAILQ
  perm_idx = jnp.where(idx >= _MAIN, perm_tail, perm_main)
  emb = _make_sc_gather()(packed, perm_idx)
  return _mlp(emb, W1, b1.reshape(1, HIDDENS), W2, b2.reshape(1, OUTPUT_DIM))
